# baseline (speedup 1.0000x reference)
import functools

import jax
import jax.numpy as jnp
from jax import lax
from jax.experimental import pallas as pl
from jax.experimental.pallas import tpu as pltpu
from jax.experimental.pallas import tpu_sc as plsc

B = 16384
C = 26
V = 100000
D = 32

N = B * C
NC = 2
NS = 16
NW = NC * NS
PER_W = N // NW
R = 1664
CHUNKS = PER_W // R
L = 16
WORDS = C * D
KPP = WORDS // L

_mesh = plsc.VectorSubcoreMesh(core_axis_name="c", subcore_axis_name="s")


@functools.partial(
    pl.kernel,
    out_type=jax.ShapeDtypeStruct((N, D), jnp.float32),
    mesh=_mesh,
    compiler_params=pltpu.CompilerParams(use_tc_tiling_on_sc=False),
    scratch_types=[
        pltpu.VMEM((R,), jnp.int32),
        pltpu.VMEM((R,), jnp.int32),
        pltpu.VMEM((WORDS,), jnp.float32),
        pltpu.VMEM((R, D), jnp.float32),
        pltpu.SemaphoreType.DMA,
    ],
)
def _embed(idx_hbm, tab_hbm, bias_hbm, off_hbm, out_hbm,
           idx_v, off_v, bias_v, rows_v, sem):
    wid = lax.axis_index("s") * NC + lax.axis_index("c")
    base = wid * PER_W
    pltpu.sync_copy(bias_hbm, bias_v)
    pltpu.sync_copy(off_hbm, off_v)

    def chunk_body(i, carry):
        start = base + i * R
        pltpu.sync_copy(idx_hbm.at[pl.ds(start, R)], idx_v)

        def add_off(k, c2):
            s = k * L
            idx_v[pl.ds(s, L)] = idx_v[pl.ds(s, L)] + off_v[pl.ds(s, L)]
            return c2

        lax.fori_loop(0, R // L, add_off, 0, unroll=4)
        pltpu.async_copy(tab_hbm.at[idx_v], rows_v, sem).wait()

        def add_bias(p, c2):
            r0 = p * C
            for k in range(KPP):
                r = r0 + k // 2
                cs = (k % 2) * L
                rows_v[r, pl.ds(cs, L)] = (
                    rows_v[r, pl.ds(cs, L)] + bias_v[pl.ds(k * L, L)])
            return c2

        lax.fori_loop(0, R // C, add_bias, 0)
        pltpu.sync_copy(rows_v, out_hbm.at[pl.ds(start, R)])
        return carry

    lax.fori_loop(0, CHUNKS, chunk_body, 0)


def kernel(indices, tables, col_table):
    idx_flat = indices.astype(jnp.int32).reshape(N)
    tab_flat = tables.reshape(C * V, D)
    bias_flat = col_table.reshape(WORDS)
    off = jnp.tile(jnp.arange(C, dtype=jnp.int32) * V, R // C)
    out = _embed(idx_flat, tab_flat, bias_flat, off)
    return out.reshape(B, C, D)


# speedup vs baseline: 3.3898x; 3.3898x over previous
import functools

import jax
import jax.numpy as jnp
from jax import lax
from jax.experimental import pallas as pl
from jax.experimental.pallas import tpu as pltpu
from jax.experimental.pallas import tpu_sc as plsc

B = 16384
C = 26
V = 100000
D = 32

NC = 2
NS = 16
NW = NC * NS
L = 16
CB = 8192

_mesh = plsc.VectorSubcoreMesh(core_axis_name="c", subcore_axis_name="s")


@functools.partial(
    pl.kernel,
    out_type=jax.ShapeDtypeStruct((C * D, B), jnp.float32),
    mesh=_mesh,
    compiler_params=pltpu.CompilerParams(needs_layout_passes=False),
    scratch_types=[
        pltpu.VMEM((V,), jnp.float32),
        pltpu.VMEM((B,), jnp.int32),
        pltpu.VMEM((CB,), jnp.float32),
        pltpu.VMEM((C * D,), jnp.float32),
        pltpu.SemaphoreType.DMA,
    ],
)
def _embed(idx_hbm, tab_hbm, col_hbm, out_hbm, vec_v, idx_v, o_v, col_v, sem):
    w = lax.axis_index("s") * NC + lax.axis_index("c")
    pltpu.sync_copy(col_hbm, col_v)

    def per_c(c, carry):
        row = c * D + w
        h = pltpu.async_copy(tab_hbm.at[row], vec_v, sem)
        pltpu.sync_copy(idx_hbm.at[c], idx_v)
        h.wait()
        bias = plsc.load_gather(col_v, [jnp.full((L,), row, jnp.int32)])

        def half(hh, cc):
            def inner(i, cc2):
                ids = idx_v[pl.ds(hh * CB + i * L, L)]
                o_v[pl.ds(i * L, L)] = plsc.load_gather(vec_v, [ids]) + bias
                return cc2

            lax.fori_loop(0, CB // L, inner, 0, unroll=8)
            pltpu.sync_copy(o_v, out_hbm.at[row, pl.ds(hh * CB, CB)])
            return cc

        lax.fori_loop(0, 2, half, 0)
        return carry

    lax.fori_loop(0, C, per_c, 0)


def kernel(indices, tables, col_table):
    idx_t = indices.astype(jnp.int32).T
    tab_t = tables.transpose(0, 2, 1).reshape(C * D, V)
    out = _embed(idx_t, tab_t, col_table.reshape(C * D))
    return out.reshape(C, D, B).transpose(2, 0, 1)


# speedup vs baseline: 6.5475x; 1.9316x over previous
import functools

import jax
import jax.numpy as jnp
from jax import lax
from jax.experimental import pallas as pl
from jax.experimental.pallas import tpu as pltpu
from jax.experimental.pallas import tpu_sc as plsc

B = 16384
C = 26
V = 100000
D = 32

NC = 2
NS = 16
NW = NC * NS
L = 16
CB = 8192

_mesh = plsc.VectorSubcoreMesh(core_axis_name="c", subcore_axis_name="s")


@functools.partial(
    pl.kernel,
    out_type=jax.ShapeDtypeStruct((C * D, B), jnp.float32),
    mesh=_mesh,
    compiler_params=pltpu.CompilerParams(needs_layout_passes=False),
    scratch_types=[
        pltpu.VMEM((V,), jnp.float32),
        pltpu.VMEM((B,), jnp.int32),
        pltpu.VMEM((CB,), jnp.float32),
        pltpu.VMEM((C * D,), jnp.float32),
        pltpu.SemaphoreType.DMA,
    ],
)
def _embed(idx_hbm, tab_hbm, col_hbm, out_hbm, vec_v, idx_v, o_v, col_v, sem):
    w = lax.axis_index("s") * NC + lax.axis_index("c")
    pltpu.sync_copy(col_hbm, col_v)

    def per_c(c, carry):
        row = c * D + w
        h = pltpu.async_copy(tab_hbm.at[row], vec_v, sem)
        pltpu.sync_copy(idx_hbm.at[c], idx_v)
        h.wait()
        bias = plsc.load_gather(col_v, [jnp.full((L,), row, jnp.int32)])

        def half(hh, cc):
            @plsc.parallel_loop(0, CB // L, unroll=8)
            def _gather(i):
                ids = idx_v[pl.ds(hh * CB + i * L, L)]
                o_v[pl.ds(i * L, L)] = plsc.load_gather(vec_v, [ids]) + bias
            pltpu.sync_copy(o_v, out_hbm.at[row, pl.ds(hh * CB, CB)])
            return cc

        lax.fori_loop(0, 2, half, 0)
        return carry

    lax.fori_loop(0, C, per_c, 0)


def kernel(indices, tables, col_table):
    idx_t = indices.astype(jnp.int32).T
    tab_t = tables.transpose(0, 2, 1).reshape(C * D, V)
    out = _embed(idx_t, tab_t, col_table.reshape(C * D))
    return out.reshape(C, D, B).transpose(2, 0, 1)
